# baseline (speedup 1.0000x reference)
import functools

import jax
import jax.numpy as jnp
from jax import lax
from jax.experimental import pallas as pl
from jax.experimental.pallas import tpu as pltpu
from jax.experimental.pallas import tpu_sc as plsc

_LB = 256
_CH = 32


def _sc_gather_rows(table, idx):
    n = idx.shape[0]
    d = table.shape[1]
    info = plsc.get_sparse_core_info()
    nw = info.num_cores * info.num_subcores
    rows_w = n // nw
    nch = rows_w // _CH
    mesh = plsc.VectorSubcoreMesh(core_axis_name="c", subcore_axis_name="s")

    @functools.partial(
        pl.kernel,
        mesh=mesh,
        out_type=jax.ShapeDtypeStruct((n, d), jnp.float32),
        scratch_types=[
            pltpu.VMEM((_CH,), jnp.int32),
            pltpu.VMEM((_CH, d), jnp.float32),
            pltpu.SemaphoreType.DMA,
        ],
    )
    def gather_k(table_hbm, idx_hbm, out_hbm, idx_v, rows_v, sem):
        wid = lax.axis_index("s") * info.num_cores + lax.axis_index("c")
        base = wid * rows_w

        def body(i, carry):
            off = base + i * _CH
            pltpu.sync_copy(idx_hbm.at[pl.ds(off, _CH)], idx_v)
            pltpu.async_copy(table_hbm.at[idx_v], rows_v, sem).wait()
            pltpu.sync_copy(rows_v, out_hbm.at[pl.ds(off, _CH)])
            return carry

        lax.fori_loop(0, nch, body, 0)

    return gather_k(table, idx)


def _tc_assemble(te3, states, mult, state_w, w8, c8, b_total, b0, prev=None):
    bc, l, d = te3.shape
    sdim = states.shape[-1]
    spread = w8.shape[0]
    a = spread - 2

    def body(te_ref, st_ref, m_ref, sw_ref, w_ref, c_ref, *rest):
        out_ref = rest[-1]
        te = te_ref[0]
        te2 = te + te
        s_emb = jnp.dot(st_ref[0], sw_ref[...],
                        preferred_element_type=jnp.float32)
        m = m_ref[0]
        out_ref[0, :, 0, :] = m[:, 0][:, None] * w_ref[0][None, :] \
            + c_ref[0][None, :] + te2
        out_ref[0, :, 1, :] = s_emb + c_ref[1][None, :] + te2
        for j in range(a):
            k = 2 + j
            out_ref[0, :, k, :] = (
                m[:, k][:, None] * w_ref[k][None, :]
                + c_ref[k][None, :] + te)
        del out_ref

    in_specs = [
        pl.BlockSpec((1, _LB, d), lambda i, j: (i, j, 0)),
        pl.BlockSpec((1, _LB, sdim), lambda i, j: (i, j, 0)),
        pl.BlockSpec((1, _LB, spread), lambda i, j: (i, j, 0)),
        pl.BlockSpec((sdim, d), lambda i, j: (0, 0)),
        pl.BlockSpec((spread, d), lambda i, j: (0, 0)),
        pl.BlockSpec((spread, d), lambda i, j: (0, 0)),
    ]
    args = [te3, states, mult, state_w, w8, c8]
    aliases = {}
    if prev is not None:
        in_specs.append(pl.BlockSpec(memory_space=pl.ANY))
        args.append(prev)
        aliases = {6: 0}

    return pl.pallas_call(
        body,
        grid=(bc, l // _LB),
        in_specs=in_specs,
        out_specs=pl.BlockSpec(
            (1, _LB, spread, d), lambda i, j: (i + b0, j, 0, 0)),
        out_shape=jax.ShapeDtypeStruct((b_total, l, spread, d), jnp.float32),
        input_output_aliases=aliases,
        compiler_params=pltpu.CompilerParams(
            dimension_semantics=("parallel", "parallel"),
        ),
    )(*args)


def kernel(states, actions, returns_to_go, time_steps, padding_mask,
           timestep_table, state_W, state_b, return_W, return_b,
           act_W, act_b, action_pos_table):
    b, l, sdim = states.shape
    a = actions.shape[-1]
    d = timestep_table.shape[1]
    spread = 2 + a

    zcol = jnp.zeros((b, l, 1), jnp.float32)
    mult = jnp.concatenate(
        [returns_to_go[..., None], zcol, actions], axis=-1)
    zrow = jnp.zeros((1, d), jnp.float32)
    w8 = jnp.concatenate([return_W[None], zrow] + [act_W[None]] * a, axis=0)
    c8 = jnp.concatenate(
        [return_b[None], state_b[None], act_b[None] + action_pos_table],
        axis=0)

    ts_flat = time_steps.astype(jnp.int32)
    half = b // 2
    te_a = _sc_gather_rows(
        timestep_table, ts_flat[:half].reshape(half * l)).reshape(half, l, d)
    te_b = _sc_gather_rows(
        timestep_table, ts_flat[half:].reshape(half * l)).reshape(half, l, d)
    out1 = _tc_assemble(te_a, states[:half], mult[:half], state_W, w8, c8,
                        b, 0)
    out = _tc_assemble(te_b, states[half:], mult[half:], state_W, w8, c8,
                       b, half, prev=out1)
    embeds = out.reshape(b, l * spread, d)
    pm = jnp.repeat(padding_mask, spread, axis=1)
    return embeds, pm


# speedup vs baseline: 1.0389x; 1.0389x over previous
import functools

import jax
import jax.numpy as jnp
from jax import lax
from jax.experimental import pallas as pl
from jax.experimental.pallas import tpu as pltpu
from jax.experimental.pallas import tpu_sc as plsc

_LB = 256
_CH = 32


def _sc_gather_rows(table, idx):
    n = idx.shape[0]
    d = table.shape[1]
    info = plsc.get_sparse_core_info()
    nw = info.num_cores * info.num_subcores
    rows_w = n // nw
    nch = rows_w // _CH
    mesh = plsc.VectorSubcoreMesh(core_axis_name="c", subcore_axis_name="s")

    @functools.partial(
        pl.kernel,
        mesh=mesh,
        out_type=jax.ShapeDtypeStruct((n, d), jnp.float32),
        scratch_types=[
            pltpu.VMEM((rows_w,), jnp.int32),
            pltpu.VMEM((_CH, d), jnp.float32),
            pltpu.VMEM((_CH, d), jnp.float32),
            pltpu.SemaphoreType.DMA,
            pltpu.SemaphoreType.DMA,
            pltpu.SemaphoreType.DMA,
            pltpu.SemaphoreType.DMA,
        ],
    )
    def gather_k(table_hbm, idx_hbm, out_hbm, idx_v, rows_v0, rows_v1,
                 gs0, gs1, ws0, ws1):
        wid = lax.axis_index("s") * info.num_cores + lax.axis_index("c")
        base = wid * rows_w
        bufs = (rows_v0, rows_v1)
        gsem = (gs0, gs1)
        wsem = (ws0, ws1)

        pltpu.sync_copy(idx_hbm.at[pl.ds(base, rows_w)], idx_v)

        def start_gather(c):
            return pltpu.async_copy(
                table_hbm.at[idx_v.at[pl.ds(c * _CH, _CH)]],
                bufs[c % 2], gsem[c % 2])

        gh = {0: start_gather(0)}
        if nch > 1:
            gh[1] = start_gather(1)
        wh = {}
        for c in range(nch):
            gh[c].wait()
            wh[c] = pltpu.async_copy(
                bufs[c % 2], out_hbm.at[pl.ds(base + c * _CH, _CH)],
                wsem[c % 2])
            if c + 2 < nch:
                wh[c].wait()
                gh[c + 2] = start_gather(c + 2)
        for c in range(max(0, nch - 2), nch):
            wh[c].wait()

    return gather_k(table, idx)


def _tc_assemble(te3, states, mult, state_w, w8, c8, b_total, b0, prev=None):
    bc, l, d = te3.shape
    sdim = states.shape[-1]
    spread = w8.shape[0]
    a = spread - 2

    def body(te_ref, st_ref, m_ref, sw_ref, w_ref, c_ref, *rest):
        out_ref = rest[-1]
        te = te_ref[0]
        te2 = te + te
        s_emb = jnp.dot(st_ref[0], sw_ref[...],
                        preferred_element_type=jnp.float32)
        m = m_ref[0]
        out_ref[0, :, 0, :] = m[:, 0][:, None] * w_ref[0][None, :] \
            + c_ref[0][None, :] + te2
        out_ref[0, :, 1, :] = s_emb + c_ref[1][None, :] + te2
        for j in range(a):
            k = 2 + j
            out_ref[0, :, k, :] = (
                m[:, k][:, None] * w_ref[k][None, :]
                + c_ref[k][None, :] + te)
        del out_ref

    in_specs = [
        pl.BlockSpec((1, _LB, d), lambda i, j: (i, j, 0)),
        pl.BlockSpec((1, _LB, sdim), lambda i, j: (i, j, 0)),
        pl.BlockSpec((1, _LB, spread), lambda i, j: (i, j, 0)),
        pl.BlockSpec((sdim, d), lambda i, j: (0, 0)),
        pl.BlockSpec((spread, d), lambda i, j: (0, 0)),
        pl.BlockSpec((spread, d), lambda i, j: (0, 0)),
    ]
    args = [te3, states, mult, state_w, w8, c8]
    aliases = {}
    if prev is not None:
        in_specs.append(pl.BlockSpec(memory_space=pl.ANY))
        args.append(prev)
        aliases = {6: 0}

    return pl.pallas_call(
        body,
        grid=(bc, l // _LB),
        in_specs=in_specs,
        out_specs=pl.BlockSpec(
            (1, _LB, spread, d), lambda i, j: (i + b0, j, 0, 0)),
        out_shape=jax.ShapeDtypeStruct((b_total, l, spread, d), jnp.float32),
        input_output_aliases=aliases,
        compiler_params=pltpu.CompilerParams(
            dimension_semantics=("parallel", "parallel"),
        ),
    )(*args)


def kernel(states, actions, returns_to_go, time_steps, padding_mask,
           timestep_table, state_W, state_b, return_W, return_b,
           act_W, act_b, action_pos_table):
    b, l, sdim = states.shape
    a = actions.shape[-1]
    d = timestep_table.shape[1]
    spread = 2 + a

    zcol = jnp.zeros((b, l, 1), jnp.float32)
    mult = jnp.concatenate(
        [returns_to_go[..., None], zcol, actions], axis=-1)
    zrow = jnp.zeros((1, d), jnp.float32)
    w8 = jnp.concatenate([return_W[None], zrow] + [act_W[None]] * a, axis=0)
    c8 = jnp.concatenate(
        [return_b[None], state_b[None], act_b[None] + action_pos_table],
        axis=0)

    te3 = _sc_gather_rows(
        timestep_table,
        time_steps.astype(jnp.int32).reshape(b * l)).reshape(b, l, d)
    out = _tc_assemble(te3, states, mult, state_W, w8, c8, b, 0)
    embeds = out.reshape(b, l * spread, d)
    pm = jnp.repeat(padding_mask, spread, axis=1)
    return embeds, pm
